# baseline (speedup 1.0000x reference)
import functools

import jax
import jax.numpy as jnp
from jax import lax
from jax.experimental import pallas as pl
from jax.experimental.pallas import tpu as pltpu
from jax.experimental.pallas import tpu_sc as plsc

NC = 2
NS = 16
NW = NC * NS
CHUNK = 128
H = 3


def _tc_hlin(x, w, b, n_acc):
    n = x.shape[0]

    def body(x_ref, w_ref, b_ref, o_ref):
        res = lax.dot_general(w_ref[...], x_ref[...],
                              (((0,), (1,)), ((), ())),
                              preferred_element_type=jnp.float32) + b_ref[...]
        o_ref[...] = jnp.pad(res, ((0, 0), (0, n_acc - n)))

    return pl.pallas_call(
        body,
        out_shape=jax.ShapeDtypeStruct((H, n_acc), jnp.float32),
    )(x, w, b)


def _tc_scale(cnt, hlin_t):
    n_acc = hlin_t.shape[1]

    def body(c_ref, hl_ref, s_ref, dinv_ref):
        deg = c_ref[0, :] + c_ref[1, :] + 1.0
        dinv = lax.rsqrt(deg)
        dinv_ref[0, :] = dinv
        s_ref[...] = dinv[None, :] * hl_ref[...]

    return pl.pallas_call(
        body,
        out_shape=[
            jax.ShapeDtypeStruct((H, n_acc), jnp.float32),
            jax.ShapeDtypeStruct((1, n_acc), jnp.float32),
        ],
    )(cnt, hlin_t)


def _tc_out(ma0, ma1, ma2, dinv, hlin_t, w_out, b_out):
    n_acc = hlin_t.shape[1]
    c = w_out.shape[1]

    def body(ma0_ref, ma1_ref, ma2_ref, dv_ref, hl_ref, w_ref, b_ref,
             h_ref, z_ref):
        dinv = dv_ref[0, :]
        for l, ma in enumerate((ma0_ref, ma1_ref, ma2_ref)):
            m_l = ma[0, :] + ma[1, :]
            h_ref[l, :] = jnp.maximum(
                dinv * m_l + dinv * dinv * hl_ref[l, :], 0.0)
        z_ref[...] = (
            lax.dot_general(w_ref[...], h_ref[...],
                            (((0,), (0,)), ((), ())),
                            preferred_element_type=jnp.float32)
            + b_ref[...]
        )

    return pl.pallas_call(
        body,
        out_shape=[
            jax.ShapeDtypeStruct((H, n_acc), jnp.float32),
            jax.ShapeDtypeStruct((c, n_acc), jnp.float32),
        ],
    )(ma0, ma1, ma2, dinv, hlin_t, w_out, b_out)


def _worker_span(nchunks, r8):
    cid = lax.axis_index("c")
    sid = lax.axis_index("s")
    wid = cid * NS + sid
    start = wid * r8
    count = jnp.clip(nchunks - start, 0, r8)
    load_start = jnp.maximum(0, jnp.minimum(start, nchunks - r8))
    joff = start - load_start
    return cid, sid, count, load_start, joff


def _load_idx_rows(ei_hbm, dst2d, base, r8, sem):
    @pl.loop(0, r8, step=8)
    def _(k):
        for b in range(8):
            pltpu.async_copy(
                ei_hbm.at[pl.ds(base + (k + b) * CHUNK, CHUNK)],
                dst2d.at[k + b], sem)
        for b in range(8):
            pltpu.make_async_copy(
                ei_hbm.at[pl.ds(base + (k + b) * CHUNK, CHUNK)],
                dst2d.at[k + b], sem).wait()


def _sc_histogram(ei_flat, zeros_rows, ones_blk, n_acc, nchunks, r8, half):
    rows_per_sub = n_acc // NS
    mesh = plsc.VectorSubcoreMesh(core_axis_name="c", subcore_axis_name="s")

    @functools.partial(
        pl.kernel,
        out_type=jax.ShapeDtypeStruct((NC, n_acc), jnp.float32),
        mesh=mesh,
        scratch_types=[
            pltpu.VMEM((r8, CHUNK), jnp.int32),
            pltpu.VMEM((CHUNK,), jnp.float32),
            pltpu.VMEM((rows_per_sub,), jnp.float32),
            pltpu.VMEM_SHARED((n_acc,), jnp.float32),
            pltpu.SemaphoreType.DMA,
        ],
    )
    def k(ei_hbm, zeros_hbm, ones_hbm, out_hbm, cols_v, ones_v, zv, acc_sh,
          hsem):
        cid, sid, count, load_start, joff = _worker_span(nchunks, r8)
        sl = pl.ds(sid * rows_per_sub, rows_per_sub)
        pltpu.sync_copy(zeros_hbm, zv)
        pltpu.sync_copy(zv, acc_sh.at[sl])
        pltpu.sync_copy(ones_hbm, ones_v)
        _load_idx_rows(ei_hbm, cols_v, half + load_start * CHUNK, r8, hsem)
        plsc.subcore_barrier()

        @pl.loop(0, r8, step=8)
        def _(j):
            for b in range(8):
                @pl.when(j + b < count)
                def _():
                    pltpu.async_copy(ones_v, acc_sh.at[cols_v.at[j + b + joff]],
                                     hsem, add=True)
            for b in range(8):
                @pl.when(j + b < count)
                def _():
                    pltpu.make_async_copy(
                        ones_v, acc_sh.at[cols_v.at[j + b + joff]], hsem).wait()

        plsc.subcore_barrier()
        pltpu.sync_copy(acc_sh.at[sl], zv)
        pltpu.sync_copy(zv, out_hbm.at[cid].at[sl])

    return k(ei_flat, zeros_rows, ones_blk)


def _sc_messages(s0, s1, s2, ei_flat, zeros_rows, n_acc, nchunks, r8, half):
    rows_per_sub = n_acc // NS
    mesh = plsc.VectorSubcoreMesh(core_axis_name="c", subcore_axis_name="s")

    @functools.partial(
        pl.kernel,
        out_type=[jax.ShapeDtypeStruct((NC, n_acc), jnp.float32)] * H,
        mesh=mesh,
        scratch_types=(
            [pltpu.VMEM((r8, CHUNK), jnp.int32)] * 2
            + [pltpu.VMEM((CHUNK,), jnp.float32)] * 6
            + [pltpu.VMEM((rows_per_sub,), jnp.float32)]
            + [pltpu.VMEM_SHARED((n_acc,), jnp.float32)] * 3
            + [pltpu.VMEM_SHARED((n_acc,), jnp.float32)] * 3
            + [pltpu.SemaphoreType.DMA] * 4
        ),
    )
    def k(s0_hbm, s1_hbm, s2_hbm, ei_hbm, zeros_hbm,
          out0_hbm, out1_hbm, out2_hbm,
          rows_v, cols_v, g00, g01, g02, g10, g11, g12, zv,
          tab0, tab1, tab2, acc0, acc1, acc2, gsem0, gsem1, ssem0, ssem1):
        outs = (out0_hbm, out1_hbm, out2_hbm)
        cid, sid, count, load_start, joff = _worker_span(nchunks, r8)
        sl = pl.ds(sid * rows_per_sub, rows_per_sub)
        s_hbm = (s0_hbm, s1_hbm, s2_hbm)
        tabs = (tab0, tab1, tab2)
        accs = (acc0, acc1, acc2)
        bufs = ((g00, g01, g02), (g10, g11, g12))
        gsems = (gsem0, gsem1)
        ssems = (ssem0, ssem1)

        for l in range(H):
            pltpu.sync_copy(s_hbm[l].at[sl], zv)
            pltpu.sync_copy(zv, tabs[l].at[sl])
        pltpu.sync_copy(zeros_hbm, zv)
        for a in accs:
            pltpu.sync_copy(zv, a.at[sl])
        _load_idx_rows(ei_hbm, rows_v, load_start * CHUNK, r8, gsem0)
        _load_idx_rows(ei_hbm, cols_v, half + load_start * CHUNK, r8, gsem1)
        plsc.subcore_barrier()

        def start_g(j, b):
            for l in range(H):
                pltpu.async_copy(tabs[l].at[rows_v.at[j + joff]], bufs[b][l],
                                 gsems[b])

        def wait_g(j, b):
            for l in range(H):
                pltpu.make_async_copy(
                    tabs[l].at[rows_v.at[j + joff]], bufs[b][l],
                    gsems[b]).wait()

        def start_s(j, b):
            for l in range(H):
                pltpu.async_copy(bufs[b][l], accs[l].at[cols_v.at[j + joff]],
                                 ssems[b], add=True)

        def wait_s(j, b):
            for l in range(H):
                pltpu.make_async_copy(
                    bufs[b][l], accs[l].at[cols_v.at[j + joff]],
                    ssems[b]).wait()

        @pl.when(0 < count)
        def _():
            start_g(0, 0)

        @pl.when(1 < count)
        def _():
            start_g(1, 1)

        @pl.loop(0, r8, step=2)
        def _(j):
            @pl.when(j < count)
            def _():
                wait_g(j, 0)
                start_s(j, 0)

            @pl.when(j + 1 < count)
            def _():
                wait_g(j + 1, 1)
                start_s(j + 1, 1)

            @pl.when(j + 2 < count)
            def _():
                wait_s(j, 0)
                start_g(j + 2, 0)

            @pl.when(j + 3 < count)
            def _():
                wait_s(j + 1, 1)
                start_g(j + 3, 1)

        for d in (2, 1):
            jt = count - d
            for b in range(2):
                @pl.when(jnp.logical_and(jt >= 0, jt % 2 == b))
                def _(jt=jt, b=b):
                    wait_s(jt, b)

        plsc.subcore_barrier()
        for l in range(H):
            pltpu.sync_copy(accs[l].at[sl], zv)
            pltpu.sync_copy(zv, outs[l].at[cid].at[sl])

    return k(s0, s1, s2, ei_flat, zeros_rows)


def kernel(x, edge_index, W_gcn, b_gcn, W_out, b_out):
    n, d = x.shape
    e = edge_index.shape[1]

    n_acc = (n // (NS * 128) + 1) * (NS * 128)

    if e % CHUNK:
        pad = CHUNK - e % CHUNK
        row = jnp.concatenate([edge_index[0], jnp.zeros((pad,), jnp.int32)])
        col = jnp.concatenate([edge_index[1], jnp.full((pad,), n, jnp.int32)])
        ei_flat = jnp.concatenate([row, col])
    else:
        ei_flat = edge_index.reshape(-1)
    half = ei_flat.shape[0] // 2
    nchunks = half // CHUNK
    chunks_per_w = -(-nchunks // NW)
    r8 = -(-chunks_per_w // 8) * 8

    zeros_rows = jnp.zeros((n_acc // NS,), jnp.float32)
    ones_blk = jnp.ones((CHUNK,), jnp.float32)

    hlin_t = _tc_hlin(x, W_gcn, b_gcn.reshape(H, 1), n_acc)
    cnt = _sc_histogram(ei_flat, zeros_rows, ones_blk, n_acc, nchunks, r8,
                        half)
    s_t, dinv = _tc_scale(cnt, hlin_t)
    ma0, ma1, ma2 = _sc_messages(s_t[0], s_t[1], s_t[2], ei_flat,
                                 zeros_rows, n_acc, nchunks, r8, half)
    h_t, z_t = _tc_out(ma0, ma1, ma2, dinv, hlin_t,
                       W_out, b_out.reshape(-1, 1))
    return (h_t[:, :n].T, z_t[:, :n].T)


# speedup vs baseline: 1.1028x; 1.1028x over previous
import functools

import jax
import jax.numpy as jnp
from jax import lax
from jax.experimental import pallas as pl
from jax.experimental.pallas import tpu as pltpu
from jax.experimental.pallas import tpu_sc as plsc

NC = 2
NS = 16
NW = NC * NS
CHUNK = 128
H = 3


def _tc_hlin(x, w, b, n_acc):
    n = x.shape[0]

    def body(x_ref, w_ref, b_ref, o_ref):
        res = lax.dot_general(w_ref[...], x_ref[...],
                              (((0,), (1,)), ((), ())),
                              preferred_element_type=jnp.float32) + b_ref[...]
        o_ref[...] = jnp.pad(res, ((0, 0), (0, n_acc - n)))

    return pl.pallas_call(
        body,
        out_shape=jax.ShapeDtypeStruct((H, n_acc), jnp.float32),
    )(x, w, b)


def _tc_scale(cnt, hlin_t):
    n_acc = hlin_t.shape[1]

    def body(c_ref, hl_ref, s_ref, dinv_ref):
        deg = c_ref[0, :] + c_ref[1, :] + 1.0
        dinv = lax.rsqrt(deg)
        dinv_ref[0, :] = dinv
        s_ref[...] = dinv[None, :] * hl_ref[...]

    return pl.pallas_call(
        body,
        out_shape=[
            jax.ShapeDtypeStruct((H, n_acc), jnp.float32),
            jax.ShapeDtypeStruct((1, n_acc), jnp.float32),
        ],
    )(cnt, hlin_t)


def _tc_out(ma0, ma1, ma2, dinv, hlin_t, w_out, b_out):
    n_acc = hlin_t.shape[1]
    c = w_out.shape[1]

    def body(ma0_ref, ma1_ref, ma2_ref, dv_ref, hl_ref, w_ref, b_ref,
             h_ref, z_ref):
        dinv = dv_ref[0, :]
        for l, ma in enumerate((ma0_ref, ma1_ref, ma2_ref)):
            m_l = ma[0, :] + ma[1, :]
            h_ref[l, :] = jnp.maximum(
                dinv * m_l + dinv * dinv * hl_ref[l, :], 0.0)
        z_ref[...] = (
            lax.dot_general(w_ref[...], h_ref[...],
                            (((0,), (0,)), ((), ())),
                            preferred_element_type=jnp.float32)
            + b_ref[...]
        )

    return pl.pallas_call(
        body,
        out_shape=[
            jax.ShapeDtypeStruct((H, n_acc), jnp.float32),
            jax.ShapeDtypeStruct((c, n_acc), jnp.float32),
        ],
    )(ma0, ma1, ma2, dinv, hlin_t, w_out, b_out)


def _worker_span(nchunks, r8):
    cid = lax.axis_index("c")
    sid = lax.axis_index("s")
    wid = cid * NS + sid
    start = wid * r8
    count = jnp.clip(nchunks - start, 0, r8)
    load_start = jnp.maximum(0, jnp.minimum(start, nchunks - r8))
    joff = start - load_start
    return cid, sid, count, load_start, joff


def _load_idx_rows(ei_hbm, dst2d, base, r8, sem):
    @pl.loop(0, r8, step=8)
    def _(k):
        for b in range(8):
            pltpu.async_copy(
                ei_hbm.at[pl.ds(base + (k + b) * CHUNK, CHUNK)],
                dst2d.at[k + b], sem)
        for b in range(8):
            pltpu.make_async_copy(
                ei_hbm.at[pl.ds(base + (k + b) * CHUNK, CHUNK)],
                dst2d.at[k + b], sem).wait()


def _sc_histogram(ei_flat, zeros_rows, ones_blk, n_acc, nchunks, r8, half):
    rows_per_sub = n_acc // NS
    mesh = plsc.VectorSubcoreMesh(core_axis_name="c", subcore_axis_name="s")

    @functools.partial(
        pl.kernel,
        out_type=[
            jax.ShapeDtypeStruct((NC, n_acc), jnp.float32),
            jax.ShapeDtypeStruct((NW * r8, CHUNK), jnp.int32),
        ],
        mesh=mesh,
        scratch_types=[
            pltpu.VMEM((r8, CHUNK), jnp.int32),
            pltpu.VMEM((CHUNK,), jnp.float32),
            pltpu.VMEM((rows_per_sub,), jnp.float32),
            pltpu.VMEM_SHARED((n_acc,), jnp.float32),
            pltpu.SemaphoreType.DMA,
        ],
    )
    def k(ei_hbm, zeros_hbm, ones_hbm, out_hbm, outc_hbm, cols_v, ones_v, zv,
          acc_sh, hsem):
        cid, sid, count, load_start, joff = _worker_span(nchunks, r8)
        wid = cid * NS + sid
        sl = pl.ds(sid * rows_per_sub, rows_per_sub)
        pltpu.sync_copy(zeros_hbm, zv)
        pltpu.sync_copy(zv, acc_sh.at[sl])
        pltpu.sync_copy(ones_hbm, ones_v)
        _load_idx_rows(ei_hbm, cols_v, half + load_start * CHUNK, r8, hsem)
        pltpu.sync_copy(cols_v, outc_hbm.at[pl.ds(wid * r8, r8)])
        plsc.subcore_barrier()

        @pl.loop(0, r8, step=8)
        def _(j):
            for b in range(8):
                @pl.when(j + b < count)
                def _():
                    pltpu.async_copy(ones_v, acc_sh.at[cols_v.at[j + b + joff]],
                                     hsem, add=True)
            for b in range(8):
                @pl.when(j + b < count)
                def _():
                    pltpu.make_async_copy(
                        ones_v, acc_sh.at[cols_v.at[j + b + joff]], hsem).wait()

        plsc.subcore_barrier()
        pltpu.sync_copy(acc_sh.at[sl], zv)
        pltpu.sync_copy(zv, out_hbm.at[cid].at[sl])

    return k(ei_flat, zeros_rows, ones_blk)


def _sc_messages(s0, s1, s2, ei_flat, cols2d, zeros_rows, n_acc, nchunks, r8):
    rows_per_sub = n_acc // NS
    mesh = plsc.VectorSubcoreMesh(core_axis_name="c", subcore_axis_name="s")

    @functools.partial(
        pl.kernel,
        out_type=[jax.ShapeDtypeStruct((NC, n_acc), jnp.float32)] * H,
        mesh=mesh,
        scratch_types=(
            [pltpu.VMEM((r8 * CHUNK,), jnp.int32),
             pltpu.VMEM((r8, CHUNK), jnp.int32)]
            + [pltpu.VMEM((CHUNK,), jnp.float32)] * 6
            + [pltpu.VMEM((rows_per_sub,), jnp.float32)]
            + [pltpu.VMEM_SHARED((n_acc,), jnp.float32)] * 3
            + [pltpu.VMEM_SHARED((n_acc,), jnp.float32)] * 3
            + [pltpu.SemaphoreType.DMA] * 4
        ),
    )
    def k(s0_hbm, s1_hbm, s2_hbm, ei_hbm, cols2d_hbm, zeros_hbm,
          out0_hbm, out1_hbm, out2_hbm,
          rows_v, cols_v, g00, g01, g02, g10, g11, g12, zv,
          tab0, tab1, tab2, acc0, acc1, acc2, gsem0, gsem1, ssem0, ssem1):
        outs = (out0_hbm, out1_hbm, out2_hbm)
        cid, sid, count, load_start, joff = _worker_span(nchunks, r8)
        wid = cid * NS + sid
        sl = pl.ds(sid * rows_per_sub, rows_per_sub)
        s_hbm = (s0_hbm, s1_hbm, s2_hbm)
        tabs = (tab0, tab1, tab2)
        accs = (acc0, acc1, acc2)
        bufs = ((g00, g01, g02), (g10, g11, g12))
        gsems = (gsem0, gsem1)
        ssems = (ssem0, ssem1)

        for l in range(H):
            pltpu.sync_copy(s_hbm[l].at[sl], zv)
            pltpu.sync_copy(zv, tabs[l].at[sl])
        pltpu.sync_copy(zeros_hbm, zv)
        for a in accs:
            pltpu.sync_copy(zv, a.at[sl])
        pltpu.sync_copy(ei_hbm.at[pl.ds(load_start * CHUNK, r8 * CHUNK)],
                        rows_v)
        pltpu.sync_copy(cols2d_hbm.at[pl.ds(wid * r8, r8)], cols_v)
        plsc.subcore_barrier()

        def ridx(j):
            return rows_v.at[pl.ds((j + joff) * CHUNK, CHUNK)]

        def start_g(j, b):
            for l in range(H):
                pltpu.async_copy(tabs[l].at[ridx(j)], bufs[b][l], gsems[b])

        def wait_g(j, b):
            for l in range(H):
                pltpu.make_async_copy(
                    tabs[l].at[ridx(j)], bufs[b][l], gsems[b]).wait()

        def start_s(j, b):
            for l in range(H):
                pltpu.async_copy(bufs[b][l], accs[l].at[cols_v.at[j + joff]],
                                 ssems[b], add=True)

        def wait_s(j, b):
            for l in range(H):
                pltpu.make_async_copy(
                    bufs[b][l], accs[l].at[cols_v.at[j + joff]],
                    ssems[b]).wait()

        @pl.when(0 < count)
        def _():
            start_g(0, 0)

        @pl.when(1 < count)
        def _():
            start_g(1, 1)

        @pl.loop(0, r8, step=2)
        def _(j):
            @pl.when(j < count)
            def _():
                wait_g(j, 0)
                start_s(j, 0)

            @pl.when(j + 1 < count)
            def _():
                wait_g(j + 1, 1)
                start_s(j + 1, 1)

            @pl.when(j + 2 < count)
            def _():
                wait_s(j, 0)
                start_g(j + 2, 0)

            @pl.when(j + 3 < count)
            def _():
                wait_s(j + 1, 1)
                start_g(j + 3, 1)

        for d in (2, 1):
            jt = count - d
            for b in range(2):
                @pl.when(jnp.logical_and(jt >= 0, jt % 2 == b))
                def _(jt=jt, b=b):
                    wait_s(jt, b)

        plsc.subcore_barrier()
        for l in range(H):
            pltpu.sync_copy(accs[l].at[sl], zv)
            pltpu.sync_copy(zv, outs[l].at[cid].at[sl])

    return k(s0, s1, s2, ei_flat, cols2d, zeros_rows)


def kernel(x, edge_index, W_gcn, b_gcn, W_out, b_out):
    n, d = x.shape
    e = edge_index.shape[1]

    n_acc = (n // (NS * 128) + 1) * (NS * 128)

    if e % CHUNK:
        pad = CHUNK - e % CHUNK
        row = jnp.concatenate([edge_index[0], jnp.zeros((pad,), jnp.int32)])
        col = jnp.concatenate([edge_index[1], jnp.full((pad,), n, jnp.int32)])
        ei_flat = jnp.concatenate([row, col])
    else:
        ei_flat = edge_index.reshape(-1)
    half = ei_flat.shape[0] // 2
    nchunks = half // CHUNK
    chunks_per_w = -(-nchunks // NW)
    r8 = -(-chunks_per_w // 8) * 8

    zeros_rows = jnp.zeros((n_acc // NS,), jnp.float32)
    ones_blk = jnp.ones((CHUNK,), jnp.float32)

    hlin_t = _tc_hlin(x, W_gcn, b_gcn.reshape(H, 1), n_acc)
    cnt, cols2d = _sc_histogram(ei_flat, zeros_rows, ones_blk, n_acc,
                                nchunks, r8, half)
    s_t, dinv = _tc_scale(cnt, hlin_t)
    ma0, ma1, ma2 = _sc_messages(s_t[0], s_t[1], s_t[2], ei_flat, cols2d,
                                 zeros_rows, n_acc, nchunks, r8)
    h_t, z_t = _tc_out(ma0, ma1, ma2, dinv, hlin_t,
                       W_out, b_out.reshape(-1, 1))
    return (h_t[:, :n].T, z_t[:, :n].T)
